# baseline (speedup 1.0000x reference)
import functools

import jax
import jax.numpy as jnp
from jax import lax
from jax.experimental import pallas as pl
from jax.experimental.pallas import tpu as pltpu
from jax.experimental.pallas import tpu_sc as plsc

_N_TOKENS = 32768
_D = 768
_E = 8
_BLK = 3072

_SC_TOKENS = 2048
_TC_TOKENS = _N_TOKENS - _SC_TOKENS
_NW = 32
_TPW = _SC_TOKENS // _NW
_C = 32
_NCHUNK = _TPW // _C
_NDC = _D // 16


def _tc_body(x_ref, wt_ref, w_out_ref, i_out_ref):
    s = jnp.dot(x_ref[...], wt_ref[...], preferred_element_type=jnp.float32)
    m = jnp.max(s, axis=1, keepdims=True)
    lane = jax.lax.broadcasted_iota(jnp.int32, s.shape, 1)
    idx = jnp.min(jnp.where(s == m, lane, _E), axis=1, keepdims=True)
    w_out_ref[...] = m
    i_out_ref[...] = idx


def _tc_router(x, wt):
    return pl.pallas_call(
        _tc_body,
        grid=(_TC_TOKENS // _BLK,),
        in_specs=[
            pl.BlockSpec((_BLK, _D), lambda i: (i, 0)),
            pl.BlockSpec((_D, _E), lambda i: (0, 0)),
        ],
        out_specs=[
            pl.BlockSpec((_BLK, 1), lambda i: (i, 0)),
            pl.BlockSpec((_BLK, 1), lambda i: (i, 0)),
        ],
        out_shape=[
            jax.ShapeDtypeStruct((_TC_TOKENS, 1), jnp.float32),
            jax.ShapeDtypeStruct((_TC_TOKENS, 1), jnp.int32),
        ],
    )(x, wt)


def _sc_body(xf_hbm, wf_hbm, wout_hbm, iout_hbm,
             xbuf, wbuf, tbuf, gw, gi, wov, iov):
    cid = lax.axis_index("c")
    sid = lax.axis_index("s")
    wid = sid * 2 + cid
    base = _TC_TOKENS + wid * _TPW

    pltpu.sync_copy(wf_hbm, wbuf)

    def _round_buf_to_bf16(buf, nvec):
        def body(k, carry):
            v = buf[pl.ds(16 * k, 16)]
            t = v * jnp.float32(65537.0)
            h = t - (t - v)
            buf[pl.ds(16 * k, 16)] = h
            return carry

        lax.fori_loop(0, nvec, body, 0)

    _round_buf_to_bf16(wbuf, _E * _D // 16)

    zeros16 = jnp.zeros((16,), jnp.float32)
    for r in range(2 * _E):
        tbuf[pl.ds(r * 16, 16)] = zeros16

    def _top1_store(acc, loc):
        s = list(acc)
        for e in range(_E):
            tbuf[pl.ds(e * 32, 16)] = s[e]
        for shift in (8, 4, 2, 1):
            for e in range(_E):
                h = tbuf[pl.ds(e * 32 + shift, 16)]
                s[e] = s[e] + h
                if shift > 1:
                    tbuf[pl.ds(e * 32, 16)] = s[e]
        best = s[0]
        bidx = jnp.zeros((16,), jnp.int32)
        for e in range(1, _E):
            g = s[e] > best
            best = jnp.where(g, s[e], best)
            bidx = jnp.where(g, jnp.full((16,), e, jnp.int32), bidx)
        r = lax.rem(loc, 16)
        gw[pl.ds(r, 16)] = best
        gi[pl.ds(r, 16)] = bidx

        @pl.when(r == 15)
        def _flush():
            wov[pl.ds(loc - 15, 16)] = gw[pl.ds(0, 16)]
            iov[pl.ds(loc - 15, 16)] = gi[pl.ds(0, 16)]

    def chunk_body(j, carry):
        tok0 = base + j * _C
        pltpu.sync_copy(xf_hbm.at[pl.ds(tok0 * _D, _C * _D)], xbuf)
        _round_buf_to_bf16(xbuf, _C * _D // 16)

        def pair_body(p, carry2):
            off0 = (2 * p) * _D
            off1 = off0 + _D
            acc0 = [jnp.zeros((16,), jnp.float32) for _ in range(_E)]
            acc1 = [jnp.zeros((16,), jnp.float32) for _ in range(_E)]
            for c in range(_NDC):
                x0 = xbuf[pl.ds(off0 + c * 16, 16)]
                x1 = xbuf[pl.ds(off1 + c * 16, 16)]
                for e in range(_E):
                    wv = wbuf[pl.ds(e * _D + c * 16, 16)]
                    acc0[e] = acc0[e] + x0 * wv
                    acc1[e] = acc1[e] + x1 * wv
            loc = j * _C + 2 * p
            _top1_store(acc0, loc)
            _top1_store(acc1, loc + 1)
            return carry2

        lax.fori_loop(0, _C // 2, pair_body, 0)
        return carry

    lax.fori_loop(0, _NCHUNK, chunk_body, 0)

    pltpu.sync_copy(wov, wout_hbm.at[pl.ds(wid * _TPW, _TPW)])
    pltpu.sync_copy(iov, iout_hbm.at[pl.ds(wid * _TPW, _TPW)])


_sc_router = functools.partial(
    pl.kernel,
    out_type=[
        jax.ShapeDtypeStruct((_SC_TOKENS,), jnp.float32),
        jax.ShapeDtypeStruct((_SC_TOKENS,), jnp.int32),
    ],
    mesh=plsc.VectorSubcoreMesh(
        core_axis_name="c", subcore_axis_name="s", num_cores=2, num_subcores=16
    ),
    scratch_types=[
        pltpu.VMEM((_C * _D,), jnp.float32),
        pltpu.VMEM((_E * _D,), jnp.float32),
        pltpu.VMEM((2 * _E * 16,), jnp.float32),
        pltpu.VMEM((32,), jnp.float32),
        pltpu.VMEM((32,), jnp.int32),
        pltpu.VMEM((_TPW,), jnp.float32),
        pltpu.VMEM((_TPW,), jnp.int32),
    ],
)(_sc_body)


def kernel(x, W):
    wt = W.T
    tc_w, tc_i = _tc_router(x, wt)
    sc_w, sc_i = _sc_router(x.reshape(-1), W.reshape(-1))
    weights = jnp.concatenate([tc_w, sc_w[:, None]], axis=0)
    indices = jnp.concatenate([tc_i, sc_i[:, None]], axis=0)
    return (weights, indices)


# speedup vs baseline: 2.7605x; 2.7605x over previous
import jax
import jax.numpy as jnp
from jax.experimental import pallas as pl

_N_TOKENS = 32768
_D = 768
_E = 8
_BLK = 4096


def _router_body(x_ref, wt_ref, w_out_ref, i_out_ref):
    s = jnp.dot(x_ref[...], wt_ref[...], preferred_element_type=jnp.float32)
    m = jnp.max(s, axis=1, keepdims=True)
    lane = jax.lax.broadcasted_iota(jnp.int32, s.shape, 1)
    idx = jnp.min(jnp.where(s == m, lane, _E), axis=1, keepdims=True)
    w_out_ref[...] = m
    i_out_ref[...] = idx


def kernel(x, W):
    wt = W.T
    grid = (_N_TOKENS // _BLK,)
    weights, indices = pl.pallas_call(
        _router_body,
        grid=grid,
        in_specs=[
            pl.BlockSpec((_BLK, _D), lambda i: (i, 0)),
            pl.BlockSpec((_D, _E), lambda i: (0, 0)),
        ],
        out_specs=[
            pl.BlockSpec((_BLK, 1), lambda i: (i, 0)),
            pl.BlockSpec((_BLK, 1), lambda i: (i, 0)),
        ],
        out_shape=[
            jax.ShapeDtypeStruct((_N_TOKENS, 1), jnp.float32),
            jax.ShapeDtypeStruct((_N_TOKENS, 1), jnp.int32),
        ],
    )(x, wt)
    return (weights, indices)


# speedup vs baseline: 2.7864x; 1.0094x over previous
import jax
import jax.numpy as jnp
from jax.experimental import pallas as pl
from jax.experimental.pallas import tpu as pltpu

_N_TOKENS = 32768
_D = 768
_E = 8
_BLK = 2048
_NBLK = _N_TOKENS // _BLK
_NBUF = 4


def _router_body(x_hbm, wt_ref, w_out_ref, i_out_ref, buf, sems):
    i = pl.program_id(0)

    def _copy(blk, slot):
        return pltpu.make_async_copy(
            x_hbm.at[pl.ds(blk * _BLK, _BLK), :],
            buf.at[slot],
            sems.at[slot],
        )

    @pl.when(i == 0)
    def _prologue():
        for s in range(_NBUF):
            _copy(s, s).start()

    slot = jax.lax.rem(i, _NBUF)
    _copy(i, slot).wait()
    s = jnp.dot(buf[slot], wt_ref[...], preferred_element_type=jnp.float32)
    m = jnp.max(s, axis=1, keepdims=True)
    lane = jax.lax.broadcasted_iota(jnp.int32, s.shape, 1)
    idx = jnp.min(jnp.where(s == m, lane, _E), axis=1, keepdims=True)
    w_out_ref[...] = m
    i_out_ref[...] = idx

    @pl.when(i + _NBUF < _NBLK)
    def _next():
        _copy(i + _NBUF, slot).start()


def kernel(x, W):
    wt = W.T
    weights, indices = pl.pallas_call(
        _router_body,
        grid=(_NBLK,),
        in_specs=[
            pl.BlockSpec(memory_space=pl.ANY),
            pl.BlockSpec((_D, _E), lambda i: (0, 0)),
        ],
        out_specs=[
            pl.BlockSpec((_BLK, 1), lambda i: (i, 0)),
            pl.BlockSpec((_BLK, 1), lambda i: (i, 0)),
        ],
        out_shape=[
            jax.ShapeDtypeStruct((_N_TOKENS, 1), jnp.float32),
            jax.ShapeDtypeStruct((_N_TOKENS, 1), jnp.int32),
        ],
        scratch_shapes=[
            pltpu.VMEM((_NBUF, _BLK, _D), jnp.float32),
            pltpu.SemaphoreType.DMA((_NBUF,)),
        ],
    )(x, wt)
    return (weights, indices)


# speedup vs baseline: 2.8326x; 1.0166x over previous
import jax
import jax.numpy as jnp
from jax.experimental import pallas as pl
from jax.experimental.pallas import tpu as pltpu

_N_TOKENS = 32768
_D = 768
_E = 8
_BLK = 1024
_NBLK = _N_TOKENS // _BLK
_NBUF = 8


def _router_body(x_hbm, wt_ref, w_out_ref, i_out_ref, buf, sems):
    i = pl.program_id(0)

    def _copy(blk, slot):
        return pltpu.make_async_copy(
            x_hbm.at[pl.ds(blk * _BLK, _BLK), :],
            buf.at[slot],
            sems.at[slot],
        )

    @pl.when(i == 0)
    def _prologue():
        for s in range(_NBUF):
            _copy(s, s).start()

    slot = jax.lax.rem(i, _NBUF)
    _copy(i, slot).wait()
    s = jnp.dot(buf[slot], wt_ref[...], preferred_element_type=jnp.float32)
    m = jnp.max(s, axis=1, keepdims=True)
    lane = jax.lax.broadcasted_iota(jnp.int32, s.shape, 1)
    idx = jnp.min(jnp.where(s == m, lane, _E), axis=1, keepdims=True)
    w_out_ref[...] = m
    i_out_ref[...] = idx

    @pl.when(i + _NBUF < _NBLK)
    def _next():
        _copy(i + _NBUF, slot).start()


def kernel(x, W):
    wt = W.T
    weights, indices = pl.pallas_call(
        _router_body,
        grid=(_NBLK,),
        in_specs=[
            pl.BlockSpec(memory_space=pl.ANY),
            pl.BlockSpec((_D, _E), lambda i: (0, 0)),
        ],
        out_specs=[
            pl.BlockSpec((_BLK, 1), lambda i: (i, 0)),
            pl.BlockSpec((_BLK, 1), lambda i: (i, 0)),
        ],
        out_shape=[
            jax.ShapeDtypeStruct((_N_TOKENS, 1), jnp.float32),
            jax.ShapeDtypeStruct((_N_TOKENS, 1), jnp.int32),
        ],
        scratch_shapes=[
            pltpu.VMEM((_NBUF, _BLK, _D), jnp.float32),
            pltpu.SemaphoreType.DMA((_NBUF,)),
        ],
    )(x, wt)
    return (weights, indices)
